# baseline (speedup 1.0000x reference)
import functools

import numpy as np
import jax
import jax.numpy as jnp
from jax import lax
from jax.experimental import pallas as pl
from jax.experimental.pallas import tpu as pltpu
from jax.experimental.pallas import tpu_sc as plsc

B = 1024
T = 200
BT = B * T
LD = 64

NC = 2
NS = 16
NW = NC * NS
PER_W = BT // NW
CHUNK = 640
NCH = PER_W // CHUNK
KSUB = CHUNK // 128

_S7 = np.zeros((7, LD, LD * 7), np.float32)
_S6 = np.zeros((6, LD, LD * 6), np.float32)
for _j in range(7):
    _S7[_j, np.arange(LD), np.arange(LD) * 7 + _j] = 1.0
for _j in range(6):
    _S6[_j, np.arange(LD), np.arange(LD) * 6 + _j] = 1.0
SPREAD7 = _S7
SPREAD6 = _S6


def _sc_gather(ke0, ke1, ke2, i0, i1, i2, se0, se1, se2, se3, sidx):
    mesh = plsc.VectorSubcoreMesh(core_axis_name="c", subcore_axis_name="s",
                                  num_cores=NC, num_subcores=NS)

    @functools.partial(
        pl.kernel,
        out_type=(
            jax.ShapeDtypeStruct((BT, LD), jnp.float32),
            jax.ShapeDtypeStruct((BT, LD), jnp.float32),
            jax.ShapeDtypeStruct((BT, LD), jnp.float32),
            jax.ShapeDtypeStruct((4, B, LD), jnp.float32),
        ),
        mesh=mesh,
        scratch_types=[
            pltpu.VMEM((CHUNK,), jnp.int32),
            pltpu.VMEM((CHUNK, LD), jnp.float32),
            pltpu.VMEM((128,), jnp.int32),
            pltpu.VMEM((128, LD), jnp.float32),
            pltpu.SemaphoreType.DMA,
        ],
        compiler_params=pltpu.CompilerParams(use_tc_tiling_on_sc=False),
    )
    def k(ke0_h, ke1_h, ke2_h, i0_h, i1_h, i2_h,
          se0_h, se1_h, se2_h, se3_h, sidx_h,
          g0_h, g1_h, g2_h, gs_h,
          idx_v, rows_v, sidx_v, srows_v, sem):
        wid = lax.axis_index("s") * NC + lax.axis_index("c")

        for tab_h, i_h, g_h in ((ke0_h, i0_h, g0_h),
                                (ke1_h, i1_h, g1_h),
                                (ke2_h, i2_h, g2_h)):
            def chunk_body(c, _, tab_h=tab_h, i_h=i_h, g_h=g_h):
                base = wid * PER_W + c * CHUNK
                pltpu.sync_copy(i_h.at[pl.ds(base, CHUNK)], idx_v)
                descs = []
                for j in range(KSUB):
                    descs.append(pltpu.async_copy(
                        tab_h.at[idx_v.at[pl.ds(j * 128, 128)]],
                        rows_v.at[pl.ds(j * 128, 128)], sem))
                for d in descs:
                    d.wait()
                pltpu.sync_copy(rows_v, g_h.at[pl.ds(base, CHUNK)])
                return 0

            lax.fori_loop(0, NCH, chunk_body, 0)

        sj = wid // 8
        ss = wid % 8
        for j, se_h in enumerate((se0_h, se1_h, se2_h, se3_h)):
            @pl.when(sj == j)
            def _(j=j, se_h=se_h):
                pltpu.sync_copy(sidx_h.at[pl.ds(j * B + ss * 128, 128)], sidx_v)
                pltpu.async_copy(se_h.at[sidx_v], srows_v, sem).wait()
                pltpu.sync_copy(srows_v, gs_h.at[j, pl.ds(ss * 128, 128)])

    return k(ke0, ke1, ke2, i0, i1, i2, se0, se1, se2, se3, sidx)


def _tc_assemble(x_flat, g0, g1, g2, A_k7, A_o7):
    N = 512
    grid = (BT // N,)

    def body(x_ref, g0_ref, g1_ref, g2_ref, ak_ref, ao_ref,
             s0_ref, s1_ref, s2_ref, ok_ref, oo_ref):
        x = x_ref[...]
        x7 = jnp.concatenate(
            [x[:, :6], jnp.full((N, 1), 1.0, jnp.float32)], axis=1)
        dot = functools.partial(jnp.dot, preferred_element_type=jnp.float32,
                                precision=lax.Precision.HIGHEST)
        ok = dot(x7, ak_ref[...])
        ok += dot(g0_ref[...], s0_ref[...])
        ok += dot(g1_ref[...], s1_ref[...])
        ok += dot(g2_ref[...], s2_ref[...])
        ok_ref[...] = ok
        oo_ref[...] = dot(x7, ao_ref[...])

    full = lambda shape: pl.BlockSpec(shape, lambda i: (0,) * len(shape))
    return pl.pallas_call(
        body,
        grid=grid,
        in_specs=[
            pl.BlockSpec((N, 13), lambda i: (i, 0)),
            pl.BlockSpec((N, LD), lambda i: (i, 0)),
            pl.BlockSpec((N, LD), lambda i: (i, 0)),
            pl.BlockSpec((N, LD), lambda i: (i, 0)),
            full((7, LD * 7)),
            full((7, LD * 6)),
            full((LD, LD * 7)),
            full((LD, LD * 7)),
            full((LD, LD * 7)),
        ],
        out_specs=[
            pl.BlockSpec((N, LD * 7), lambda i: (i, 0)),
            pl.BlockSpec((N, LD * 6), lambda i: (i, 0)),
        ],
        out_shape=[
            jax.ShapeDtypeStruct((BT, LD * 7), jnp.float32),
            jax.ShapeDtypeStruct((BT, LD * 6), jnp.float32),
        ],
    )(x_flat, g0, g1, g2, A_k7, A_o7,
      jnp.asarray(SPREAD7[4]), jnp.asarray(SPREAD7[5]), jnp.asarray(SPREAD7[6]))


def kernel(inputs, se0, se1, se2, se3, ke0, ke1, ke2, Wr, br, Wo, bo):
    x_flat = inputs.reshape(BT, 13)

    kidx = x_flat[:, 10:13].astype(jnp.int32)
    i0 = kidx[:, 0]
    i1 = kidx[:, 1]
    i2 = kidx[:, 2]
    sidx = inputs[:, 0, 6:10].astype(jnp.int32).T.reshape(4 * B)

    g0, g1, g2, gs = _sc_gather(ke0, ke1, ke2, i0, i1, i2,
                                se0, se1, se2, se3, sidx)

    s7 = jnp.asarray(SPREAD7[:4])
    s6 = jnp.asarray(SPREAD6)
    A_k7 = jnp.concatenate([
        jnp.einsum("jl,jlc->jc", Wr, s7),
        jnp.zeros((2, LD * 7), jnp.float32),
        jnp.einsum("jl,jlc->c", br, s7)[None],
    ], axis=0)
    A_o7 = jnp.concatenate([
        jnp.einsum("jl,jlc->jc", Wo, s6),
        jnp.einsum("jl,jlc->c", bo, s6)[None],
    ], axis=0)

    known_flat, obs_flat = _tc_assemble(x_flat, g0, g1, g2, A_k7, A_o7)

    return (gs.transpose(1, 0, 2),
            known_flat.reshape(B, T, LD, 7),
            obs_flat.reshape(B, T, LD, 6))


# speedup vs baseline: 6.3257x; 6.3257x over previous
import functools

import numpy as np
import jax
import jax.numpy as jnp
from jax import lax
from jax.experimental import pallas as pl
from jax.experimental.pallas import tpu as pltpu
from jax.experimental.pallas import tpu_sc as plsc

B = 1024
T = 200
BT = B * T
LD = 64
KVOCAB = (100000, 1000, 52)
SVOCAB = (100000, 100000, 1000, 100)

NC = 2
NS = 16
NW = NC * NS
PER_W = BT // NW
CHUNK = 640
NCH = PER_W // CHUNK
KSUB = CHUNK // 128


HB = B // 2
ROWS = T * HB
PER_WH = ROWS // NW
NCHH = PER_WH // CHUNK


def _sc_gather(ke0, ke1, ke2, iA0, iB0, iA1, iB1, iA2, iB2,
               sef0, sef1, sef2, sef3, sidx64):
    mesh = plsc.VectorSubcoreMesh(core_axis_name="c", subcore_axis_name="s",
                                  num_cores=NC, num_subcores=NS)

    @functools.partial(
        pl.kernel,
        out_type=(
            jax.ShapeDtypeStruct((ROWS, 128), jnp.float32),
            jax.ShapeDtypeStruct((ROWS, 128), jnp.float32),
            jax.ShapeDtypeStruct((ROWS, 128), jnp.float32),
            jax.ShapeDtypeStruct((4 * LD * B,), jnp.float32),
        ),
        mesh=mesh,
        scratch_types=[
            pltpu.VMEM((CHUNK,), jnp.int32),
            pltpu.VMEM((CHUNK, LD), jnp.float32),
            pltpu.VMEM((B,), jnp.int32),
            pltpu.VMEM((B,), jnp.float32),
            pltpu.SemaphoreType.DMA,
        ],
        compiler_params=pltpu.CompilerParams(use_tc_tiling_on_sc=False),
    )
    def k(ke0_h, ke1_h, ke2_h, iA0_h, iB0_h, iA1_h, iB1_h, iA2_h, iB2_h,
          sef0_h, sef1_h, sef2_h, sef3_h, sidx_h,
          g0_h, g1_h, g2_h, gs_h,
          idx_v, rows_v, sidx_v, srow_v, sem):
        wid = lax.axis_index("s") * NC + lax.axis_index("c")

        for tab_h, i_h, g_h, cofs in ((ke0_h, iA0_h, g0_h, 0),
                                      (ke0_h, iB0_h, g0_h, LD),
                                      (ke1_h, iA1_h, g1_h, 0),
                                      (ke1_h, iB1_h, g1_h, LD),
                                      (ke2_h, iA2_h, g2_h, 0),
                                      (ke2_h, iB2_h, g2_h, LD)):
            def chunk_body(c, _, tab_h=tab_h, i_h=i_h, g_h=g_h, cofs=cofs):
                base = wid * PER_WH + c * CHUNK
                pltpu.sync_copy(i_h.at[pl.ds(base, CHUNK)], idx_v)
                descs = []
                for j in range(KSUB):
                    descs.append(pltpu.async_copy(
                        tab_h.at[idx_v.at[pl.ds(j * 128, 128)]],
                        rows_v.at[pl.ds(j * 128, 128)], sem))
                for d in descs:
                    d.wait()
                pltpu.sync_copy(rows_v,
                                g_h.at[pl.ds(base, CHUNK), pl.ds(cofs, LD)])
                return 0

            lax.fori_loop(0, NCHH, chunk_body, 0)

        sj = wid // 8
        l0 = (wid % 8) * 8
        for j, sef_h in enumerate((sef0_h, sef1_h, sef2_h, sef3_h)):
            @pl.when(sj == j)
            def _(j=j, sef_h=sef_h):
                def srow_body(k_, _, j=j, sef_h=sef_h):
                    off = (j * LD + l0 + k_) * B
                    pltpu.sync_copy(sidx_h.at[pl.ds(off, B)], sidx_v)
                    sdescs = []
                    for m in range(B // 128):
                        sdescs.append(pltpu.async_copy(
                            sef_h.at[sidx_v.at[pl.ds(m * 128, 128)]],
                            srow_v.at[pl.ds(m * 128, 128)], sem))
                    for d in sdescs:
                        d.wait()
                    pltpu.sync_copy(srow_v, gs_h.at[pl.ds(off, B)])
                    return 0

                lax.fori_loop(0, 8, srow_body, 0)

    return k(ke0, ke1, ke2, iA0, iB0, iA1, iB1, iA2, iB2,
             sef0, sef1, sef2, sef3, sidx64)


def _tc_assemble(xpad, g0, g1, g2, wk, bk, wo, bob, eye):
    def body(x_ref, g0_ref, g1_ref, g2_ref, wk_ref, bk_ref, wo_ref, bo_ref,
             eye_ref, ok_ref, oo_ref):
        x = x_ref[0]
        xk = jnp.broadcast_to(x[0:4][:, None, :], (4, LD, B)).reshape(4 * LD, B)
        ok_ref[0, 0:4 * LD, :] = xk * wk_ref[...] + bk_ref[...]
        for i, g_ref in enumerate((g0_ref, g1_ref, g2_ref)):
            g = g_ref[0]
            r0, r1 = (4 + i) * LD, (5 + i) * LD
            ok_ref[0, r0:r1, 0:HB] = lax.dot_general(
                eye_ref[...], g[:, 0:LD], (((1,), (1,)), ((), ())),
                preferred_element_type=jnp.float32)
            ok_ref[0, r0:r1, HB:B] = lax.dot_general(
                eye_ref[...], g[:, LD:128], (((1,), (1,)), ((), ())),
                preferred_element_type=jnp.float32)
        xo = jnp.broadcast_to(x[0:6][:, None, :], (6, LD, B)).reshape(6 * LD, B)
        oo_ref[0] = xo * wo_ref[...] + bo_ref[...]

    full = lambda shape: pl.BlockSpec(shape, lambda t: (0,) * len(shape))
    return pl.pallas_call(
        body,
        grid=(T,),
        in_specs=[
            pl.BlockSpec((1, 8, B), lambda t: (t, 0, 0)),
            pl.BlockSpec((1, HB, 128), lambda t: (t, 0, 0)),
            pl.BlockSpec((1, HB, 128), lambda t: (t, 0, 0)),
            pl.BlockSpec((1, HB, 128), lambda t: (t, 0, 0)),
            full((4 * LD, 1)),
            full((4 * LD, 1)),
            full((6 * LD, 1)),
            full((6 * LD, 1)),
            full((LD, LD)),
        ],
        out_specs=[
            pl.BlockSpec((1, 7 * LD, B), lambda t: (t, 0, 0)),
            pl.BlockSpec((1, 6 * LD, B), lambda t: (t, 0, 0)),
        ],
        out_shape=[
            jax.ShapeDtypeStruct((T, 7 * LD, B), jnp.float32),
            jax.ShapeDtypeStruct((T, 6 * LD, B), jnp.float32),
        ],
    )(xpad, g0, g1, g2, wk, bk, wo, bob, eye)


def kernel(inputs, se0, se1, se2, se3, ke0, ke1, ke2, Wr, br, Wo, bo):
    inT = jnp.transpose(inputs, (1, 2, 0))
    xpad = jnp.concatenate(
        [inT[:, 0:6, :], jnp.zeros((T, 2, B), jnp.float32)], axis=1)

    kidxT = inT[:, 10:13, :].astype(jnp.int32)
    iA0 = kidxT[:, 0, 0:HB].reshape(ROWS)
    iB0 = kidxT[:, 0, HB:B].reshape(ROWS)
    iA1 = kidxT[:, 1, 0:HB].reshape(ROWS)
    iB1 = kidxT[:, 1, HB:B].reshape(ROWS)
    iA2 = kidxT[:, 2, 0:HB].reshape(ROWS)
    iB2 = kidxT[:, 2, HB:B].reshape(ROWS)

    sidxb = inputs[:, 0, 6:10].astype(jnp.int32)
    planes = [
        (jnp.arange(LD, dtype=jnp.int32)[:, None] * SVOCAB[j] + sidxb[None, :, j])
        for j in range(4)
    ]
    sidx64 = jnp.stack(planes, axis=0).reshape(4 * LD * B)

    sef = [t.T.reshape(-1) for t in (se0, se1, se2, se3)]

    g0, g1, g2, gs = _sc_gather(ke0, ke1, ke2, iA0, iB0, iA1, iB1, iA2, iB2,
                                sef[0], sef[1], sef[2], sef[3], sidx64)

    wk = Wr.reshape(4 * LD, 1)
    bk = br.reshape(4 * LD, 1)
    wo = Wo.reshape(6 * LD, 1)
    bob = bo.reshape(6 * LD, 1)
    eye = jnp.asarray(np.eye(LD, dtype=np.float32))

    ok_p, oo_p = _tc_assemble(xpad,
                              g0.reshape(T, HB, 128),
                              g1.reshape(T, HB, 128),
                              g2.reshape(T, HB, 128),
                              wk, bk, wo, bob, eye)

    static = jnp.transpose(gs.reshape(4, LD, B), (2, 0, 1))
    known = jnp.transpose(ok_p.reshape(T, 7, LD, B), (3, 0, 2, 1))
    observed = jnp.transpose(oo_p.reshape(T, 6, LD, B), (3, 0, 2, 1))
    return (static, known, observed)


# speedup vs baseline: 6.7043x; 1.0599x over previous
import functools

import numpy as np
import jax
import jax.numpy as jnp
from jax import lax
from jax.experimental import pallas as pl
from jax.experimental.pallas import tpu as pltpu
from jax.experimental.pallas import tpu_sc as plsc

B = 1024
T = 200
BT = B * T
LD = 64
SVOCAB = (100000, 100000, 1000, 100)

NC = 2
NS = 16
NW = NC * NS

NSTAGE = 5
TS = T // NSTAGE
HB = B // 2
ROWS_S = TS * HB
CHUNK = ROWS_S // NW
KSUB = CHUNK // 128


def _sc_stage(with_static, ke0, ke1, ke2, idxs, sefs, sidx64):
    mesh = plsc.VectorSubcoreMesh(core_axis_name="c", subcore_axis_name="s",
                                  num_cores=NC, num_subcores=NS)
    gshape = jax.ShapeDtypeStruct((ROWS_S, 128), jnp.float32)
    out_type = [gshape, gshape, gshape]
    if with_static:
        out_type.append(jax.ShapeDtypeStruct((4 * LD * B,), jnp.float32))

    @functools.partial(
        pl.kernel,
        out_type=tuple(out_type),
        mesh=mesh,
        scratch_types=[
            pltpu.VMEM((CHUNK,), jnp.int32),
            pltpu.VMEM((CHUNK,), jnp.int32),
            pltpu.VMEM((CHUNK,), jnp.int32),
            pltpu.VMEM((CHUNK, LD), jnp.float32),
            pltpu.VMEM((CHUNK, LD), jnp.float32),
            pltpu.VMEM((CHUNK, LD), jnp.float32),
            pltpu.VMEM((B,), jnp.int32),
            pltpu.VMEM((B,), jnp.float32),
            pltpu.SemaphoreType.DMA,
            pltpu.SemaphoreType.DMA,
            pltpu.SemaphoreType.DMA,
        ],
        compiler_params=pltpu.CompilerParams(use_tc_tiling_on_sc=False),
    )
    def k(*args):
        (ke0_h, ke1_h, ke2_h, iA0_h, iB0_h, iA1_h, iB1_h, iA2_h, iB2_h,
         sef0_h, sef1_h, sef2_h, sef3_h, sidx_h) = args[:14]
        if with_static:
            g0_h, g1_h, g2_h, gs_h = args[14:18]
            scratch = args[18:]
        else:
            g0_h, g1_h, g2_h = args[14:17]
            gs_h = None
            scratch = args[17:]
        (i0_v, i1_v, i2_v, r0_v, r1_v, r2_v, sidx_v, srow_v,
         sem_i, sem_g, sem_w) = scratch
        wid = lax.axis_index("s") * NC + lax.axis_index("c")
        base = wid * CHUNK
        tabs = (ke0_h, ke1_h, ke2_h)
        gouts = (g0_h, g1_h, g2_h)
        ivs = (i0_v, i1_v, i2_v)
        rvs = (r0_v, r1_v, r2_v)

        wdescs = []
        for half, (cofs, ihs) in enumerate((
                (0, (iA0_h, iA1_h, iA2_h)),
                (LD, (iB0_h, iB1_h, iB2_h)))):
            idescs = [pltpu.async_copy(ihs[i].at[pl.ds(base, CHUNK)],
                                       ivs[i], sem_i) for i in range(3)]
            for d in idescs:
                d.wait()
            for d in wdescs:
                d.wait()
            gdescs = []
            for i in range(3):
                for j in range(KSUB):
                    gdescs.append(pltpu.async_copy(
                        tabs[i].at[ivs[i].at[pl.ds(j * 128, 128)]],
                        rvs[i].at[pl.ds(j * 128, 128)], sem_g))
            for d in gdescs:
                d.wait()
            wdescs = [pltpu.async_copy(
                rvs[i], gouts[i].at[pl.ds(base, CHUNK), pl.ds(cofs, LD)],
                sem_w) for i in range(3)]
        for d in wdescs:
            d.wait()

        if with_static:
            sj = wid // 8
            l0 = (wid % 8) * 8
            for j, sef_h in enumerate((sef0_h, sef1_h, sef2_h, sef3_h)):
                @pl.when(sj == j)
                def _(j=j, sef_h=sef_h):
                    def srow_body(k_, _, j=j, sef_h=sef_h):
                        off = (j * LD + l0 + k_) * B
                        pltpu.sync_copy(sidx_h.at[pl.ds(off, B)], sidx_v)
                        sdescs = []
                        for m in range(B // 128):
                            sdescs.append(pltpu.async_copy(
                                sef_h.at[sidx_v.at[pl.ds(m * 128, 128)]],
                                srow_v.at[pl.ds(m * 128, 128)], sem_g))
                        for d in sdescs:
                            d.wait()
                        pltpu.sync_copy(srow_v, gs_h.at[pl.ds(off, B)])
                        return 0

                    lax.fori_loop(0, 8, srow_body, 0)

    return k(ke0, ke1, ke2, *idxs, *sefs, sidx64)


def _tc_stage(s, prev, xpad, g0, g1, g2, wk, bk, wo, bob, eye):
    def body(*refs):
        if prev is None:
            (x_ref, g0_ref, g1_ref, g2_ref,
             wk_ref, bk_ref, wo_ref, bo_ref, eye_ref, ok_ref, oo_ref) = refs
        else:
            (_, _, x_ref, g0_ref, g1_ref, g2_ref,
             wk_ref, bk_ref, wo_ref, bo_ref, eye_ref, ok_ref, oo_ref) = refs
        x = x_ref[0]
        xk = jnp.broadcast_to(x[0:4][:, None, :], (4, LD, B)).reshape(4 * LD, B)
        ok_ref[0, 0:4 * LD, :] = xk * wk_ref[...] + bk_ref[...]
        for i, g_ref in enumerate((g0_ref, g1_ref, g2_ref)):
            g = g_ref[0]
            r0, r1 = (4 + i) * LD, (5 + i) * LD
            ok_ref[0, r0:r1, 0:HB] = lax.dot_general(
                eye_ref[...], g[:, 0:LD], (((1,), (1,)), ((), ())),
                preferred_element_type=jnp.float32)
            ok_ref[0, r0:r1, HB:B] = lax.dot_general(
                eye_ref[...], g[:, LD:128], (((1,), (1,)), ((), ())),
                preferred_element_type=jnp.float32)
        xo = jnp.broadcast_to(x[0:6][:, None, :], (6, LD, B)).reshape(6 * LD, B)
        oo_ref[0] = xo * wo_ref[...] + bo_ref[...]

    full = lambda shape: pl.BlockSpec(shape, lambda t: (0,) * len(shape))
    hbm = pl.BlockSpec(memory_space=pltpu.MemorySpace.HBM)
    alias_specs = [] if prev is None else [hbm, hbm]
    alias_args = () if prev is None else (prev[0], prev[1])
    aliases = {} if prev is None else {0: 0, 1: 1}
    return pl.pallas_call(
        body,
        grid=(TS,),
        in_specs=alias_specs + [
            pl.BlockSpec((1, 8, B), lambda t, s=s: (s * TS + t, 0, 0)),
            pl.BlockSpec((1, HB, 128), lambda t: (t, 0, 0)),
            pl.BlockSpec((1, HB, 128), lambda t: (t, 0, 0)),
            pl.BlockSpec((1, HB, 128), lambda t: (t, 0, 0)),
            full((4 * LD, 1)),
            full((4 * LD, 1)),
            full((6 * LD, 1)),
            full((6 * LD, 1)),
            full((LD, LD)),
        ],
        out_specs=[
            pl.BlockSpec((1, 7 * LD, B), lambda t, s=s: (s * TS + t, 0, 0)),
            pl.BlockSpec((1, 6 * LD, B), lambda t, s=s: (s * TS + t, 0, 0)),
        ],
        out_shape=[
            jax.ShapeDtypeStruct((T, 7 * LD, B), jnp.float32),
            jax.ShapeDtypeStruct((T, 6 * LD, B), jnp.float32),
        ],
        input_output_aliases=aliases,
    )(*alias_args, xpad, g0, g1, g2, wk, bk, wo, bob, eye)


def kernel(inputs, se0, se1, se2, se3, ke0, ke1, ke2, Wr, br, Wo, bo):
    inT = jnp.transpose(inputs, (1, 2, 0))
    xpad = jnp.concatenate(
        [inT[:, 0:6, :], jnp.zeros((T, 2, B), jnp.float32)], axis=1)

    kidxT = inT[:, 10:13, :].astype(jnp.int32)

    sidxb = inputs[:, 0, 6:10].astype(jnp.int32)
    planes = [
        (jnp.arange(LD, dtype=jnp.int32)[:, None] * SVOCAB[j] + sidxb[None, :, j])
        for j in range(4)
    ]
    sidx64 = jnp.stack(planes, axis=0).reshape(4 * LD * B)

    sefs = [t.T.reshape(-1) for t in (se0, se1, se2, se3)]

    wk = Wr.reshape(4 * LD, 1)
    bk = br.reshape(4 * LD, 1)
    wo = Wo.reshape(6 * LD, 1)
    bob = bo.reshape(6 * LD, 1)
    eye = jnp.asarray(np.eye(LD, dtype=np.float32))

    stage_g = []
    gs = None
    for s in range(NSTAGE):
        ks = kidxT[s * TS:(s + 1) * TS]
        idxs = []
        for i in range(3):
            idxs.append(ks[:, i, 0:HB].reshape(ROWS_S))
            idxs.append(ks[:, i, HB:B].reshape(ROWS_S))
        outs = _sc_stage(s == NSTAGE - 1, ke0, ke1, ke2, idxs, sefs, sidx64)
        if s == NSTAGE - 1:
            g0s, g1s, g2s, gs = outs
        else:
            g0s, g1s, g2s = outs
        stage_g.append((g0s.reshape(TS, HB, 128),
                        g1s.reshape(TS, HB, 128),
                        g2s.reshape(TS, HB, 128)))

    prev = None
    for s in range(NSTAGE):
        g0s, g1s, g2s = stage_g[s]
        prev = _tc_stage(s, prev, xpad, g0s, g1s, g2s, wk, bk, wo, bob, eye)
    ok_p, oo_p = prev

    static = jnp.transpose(gs.reshape(4, LD, B), (2, 0, 1))
    known = jnp.transpose(ok_p.reshape(T, 7, LD, B), (3, 0, 2, 1))
    observed = jnp.transpose(oo_p.reshape(T, 6, LD, B), (3, 0, 2, 1))
    return (static, known, observed)


# speedup vs baseline: 6.7271x; 1.0034x over previous
import functools

import numpy as np
import jax
import jax.numpy as jnp
from jax import lax
from jax.experimental import pallas as pl
from jax.experimental.pallas import tpu as pltpu
from jax.experimental.pallas import tpu_sc as plsc

B = 1024
T = 200
BT = B * T
LD = 64
SVOCAB = (100000, 100000, 1000, 100)

NC = 2
NS = 16
NW = NC * NS

NSTAGE = 5
TS = T // NSTAGE
HB = B // 2
ROWS_S = TS * HB
CHUNK = ROWS_S // NW
KSUB = CHUNK // 128


def _sc_stage(with_static, ke0, ke1, ke2, idxs, sefs, sidx64):
    mesh = plsc.VectorSubcoreMesh(core_axis_name="c", subcore_axis_name="s",
                                  num_cores=NC, num_subcores=NS)
    gshape = jax.ShapeDtypeStruct((ROWS_S, 128), jnp.float32)
    out_type = [gshape, gshape, gshape]
    if with_static:
        out_type.append(jax.ShapeDtypeStruct((4 * LD * B,), jnp.float32))

    @functools.partial(
        pl.kernel,
        out_type=tuple(out_type),
        mesh=mesh,
        scratch_types=[
            pltpu.VMEM((CHUNK,), jnp.int32),
            pltpu.VMEM((CHUNK,), jnp.int32),
            pltpu.VMEM((CHUNK,), jnp.int32),
            pltpu.VMEM((CHUNK, LD), jnp.float32),
            pltpu.VMEM((CHUNK, LD), jnp.float32),
            pltpu.VMEM((CHUNK, LD), jnp.float32),
            pltpu.VMEM((B,), jnp.int32),
            pltpu.VMEM((B,), jnp.float32),
            pltpu.SemaphoreType.DMA,
            pltpu.SemaphoreType.DMA,
            pltpu.SemaphoreType.DMA,
        ],
        compiler_params=pltpu.CompilerParams(use_tc_tiling_on_sc=False),
    )
    def k(*args):
        (ke0_h, ke1_h, ke2_h, iA0_h, iB0_h, iA1_h, iB1_h, iA2_h, iB2_h,
         sef0_h, sef1_h, sef2_h, sef3_h, sidx_h) = args[:14]
        if with_static:
            g0_h, g1_h, g2_h, gs_h = args[14:18]
            scratch = args[18:]
        else:
            g0_h, g1_h, g2_h = args[14:17]
            gs_h = None
            scratch = args[17:]
        (i0_v, i1_v, i2_v, r0_v, r1_v, r2_v, sidx_v, srow_v,
         sem_i, sem_g, sem_w) = scratch
        wid = lax.axis_index("s") * NC + lax.axis_index("c")
        base = wid * CHUNK
        tabs = (ke0_h, ke1_h, ke2_h)
        gouts = (g0_h, g1_h, g2_h)
        ivs = (i0_v, i1_v, i2_v)
        rvs = (r0_v, r1_v, r2_v)

        wdescs = []
        for half, (cofs, ihs) in enumerate((
                (0, (iA0_h, iA1_h, iA2_h)),
                (LD, (iB0_h, iB1_h, iB2_h)))):
            idescs = [pltpu.async_copy(ihs[i].at[pl.ds(base, CHUNK)],
                                       ivs[i], sem_i) for i in range(3)]
            for d in idescs:
                d.wait()
            for d in wdescs:
                d.wait()
            gdescs = []
            for i in range(3):
                gdescs.append(pltpu.async_copy(
                    tabs[i].at[ivs[i]], rvs[i], sem_g))
            for d in gdescs:
                d.wait()
            wdescs = [pltpu.async_copy(
                rvs[i], gouts[i].at[pl.ds(base, CHUNK), pl.ds(cofs, LD)],
                sem_w) for i in range(3)]
        for d in wdescs:
            d.wait()

        if with_static:
            sj = wid // 8
            l0 = (wid % 8) * 8
            for j, sef_h in enumerate((sef0_h, sef1_h, sef2_h, sef3_h)):
                @pl.when(sj == j)
                def _(j=j, sef_h=sef_h):
                    def srow_body(k_, _, j=j, sef_h=sef_h):
                        off = (j * LD + l0 + k_) * B
                        pltpu.sync_copy(sidx_h.at[pl.ds(off, B)], sidx_v)
                        sdescs = []
                        for m in range(B // 128):
                            sdescs.append(pltpu.async_copy(
                                sef_h.at[sidx_v.at[pl.ds(m * 128, 128)]],
                                srow_v.at[pl.ds(m * 128, 128)], sem_g))
                        for d in sdescs:
                            d.wait()
                        pltpu.sync_copy(srow_v, gs_h.at[pl.ds(off, B)])
                        return 0

                    lax.fori_loop(0, 8, srow_body, 0)

    return k(ke0, ke1, ke2, *idxs, *sefs, sidx64)


def _tc_stage(s, prev, xpad, g0, g1, g2, wk, bk, wo, bob, eye):
    def body(*refs):
        if prev is None:
            (x_ref, g0_ref, g1_ref, g2_ref,
             wk_ref, bk_ref, wo_ref, bo_ref, eye_ref, ok_ref, oo_ref) = refs
        else:
            (_, _, x_ref, g0_ref, g1_ref, g2_ref,
             wk_ref, bk_ref, wo_ref, bo_ref, eye_ref, ok_ref, oo_ref) = refs
        x = x_ref[0]
        xk = jnp.broadcast_to(x[0:4][:, None, :], (4, LD, B)).reshape(4 * LD, B)
        ok_ref[0, 0:4 * LD, :] = xk * wk_ref[...] + bk_ref[...]
        for i, g_ref in enumerate((g0_ref, g1_ref, g2_ref)):
            g = g_ref[0]
            r0, r1 = (4 + i) * LD, (5 + i) * LD
            ok_ref[0, r0:r1, 0:HB] = lax.dot_general(
                eye_ref[...], g[:, 0:LD], (((1,), (1,)), ((), ())),
                preferred_element_type=jnp.float32)
            ok_ref[0, r0:r1, HB:B] = lax.dot_general(
                eye_ref[...], g[:, LD:128], (((1,), (1,)), ((), ())),
                preferred_element_type=jnp.float32)
        xo = jnp.broadcast_to(x[0:6][:, None, :], (6, LD, B)).reshape(6 * LD, B)
        oo_ref[0] = xo * wo_ref[...] + bo_ref[...]

    full = lambda shape: pl.BlockSpec(shape, lambda t: (0,) * len(shape))
    hbm = pl.BlockSpec(memory_space=pltpu.MemorySpace.HBM)
    alias_specs = [] if prev is None else [hbm, hbm]
    alias_args = () if prev is None else (prev[0], prev[1])
    aliases = {} if prev is None else {0: 0, 1: 1}
    return pl.pallas_call(
        body,
        grid=(TS,),
        in_specs=alias_specs + [
            pl.BlockSpec((1, 8, B), lambda t, s=s: (s * TS + t, 0, 0)),
            pl.BlockSpec((1, HB, 128), lambda t: (t, 0, 0)),
            pl.BlockSpec((1, HB, 128), lambda t: (t, 0, 0)),
            pl.BlockSpec((1, HB, 128), lambda t: (t, 0, 0)),
            full((4 * LD, 1)),
            full((4 * LD, 1)),
            full((6 * LD, 1)),
            full((6 * LD, 1)),
            full((LD, LD)),
        ],
        out_specs=[
            pl.BlockSpec((1, 7 * LD, B), lambda t, s=s: (s * TS + t, 0, 0)),
            pl.BlockSpec((1, 6 * LD, B), lambda t, s=s: (s * TS + t, 0, 0)),
        ],
        out_shape=[
            jax.ShapeDtypeStruct((T, 7 * LD, B), jnp.float32),
            jax.ShapeDtypeStruct((T, 6 * LD, B), jnp.float32),
        ],
        input_output_aliases=aliases,
    )(*alias_args, xpad, g0, g1, g2, wk, bk, wo, bob, eye)


def kernel(inputs, se0, se1, se2, se3, ke0, ke1, ke2, Wr, br, Wo, bo):
    inT = jnp.transpose(inputs, (1, 2, 0))
    xpad = jnp.concatenate(
        [inT[:, 0:6, :], jnp.zeros((T, 2, B), jnp.float32)], axis=1)

    kidxT = inT[:, 10:13, :].astype(jnp.int32)

    sidxb = inputs[:, 0, 6:10].astype(jnp.int32)
    planes = [
        (jnp.arange(LD, dtype=jnp.int32)[:, None] * SVOCAB[j] + sidxb[None, :, j])
        for j in range(4)
    ]
    sidx64 = jnp.stack(planes, axis=0).reshape(4 * LD * B)

    sefs = [t.T.reshape(-1) for t in (se0, se1, se2, se3)]

    wk = Wr.reshape(4 * LD, 1)
    bk = br.reshape(4 * LD, 1)
    wo = Wo.reshape(6 * LD, 1)
    bob = bo.reshape(6 * LD, 1)
    eye = jnp.asarray(np.eye(LD, dtype=np.float32))

    stage_g = []
    gs = None
    for s in range(NSTAGE):
        ks = kidxT[s * TS:(s + 1) * TS]
        idxs = []
        for i in range(3):
            idxs.append(ks[:, i, 0:HB].reshape(ROWS_S))
            idxs.append(ks[:, i, HB:B].reshape(ROWS_S))
        outs = _sc_stage(s == NSTAGE - 1, ke0, ke1, ke2, idxs, sefs, sidx64)
        if s == NSTAGE - 1:
            g0s, g1s, g2s, gs = outs
        else:
            g0s, g1s, g2s = outs
        stage_g.append((g0s.reshape(TS, HB, 128),
                        g1s.reshape(TS, HB, 128),
                        g2s.reshape(TS, HB, 128)))

    prev = None
    for s in range(NSTAGE):
        g0s, g1s, g2s = stage_g[s]
        prev = _tc_stage(s, prev, xpad, g0s, g1s, g2s, wk, bk, wo, bob, eye)
    ok_p, oo_p = prev

    static = jnp.transpose(gs.reshape(4, LD, B), (2, 0, 1))
    known = jnp.transpose(ok_p.reshape(T, 7, LD, B), (3, 0, 2, 1))
    observed = jnp.transpose(oo_p.reshape(T, 6, LD, B), (3, 0, 2, 1))
    return (static, known, observed)
